# baseline (speedup 1.0000x reference)
import math

import jax
import jax.numpy as jnp
import numpy as np
from jax.experimental import pallas as pl
from jax.experimental.pallas import tpu as pltpu

_PS = 65
_NB_ANG = 8
_NB_SP = 4
_CLIP = 0.2
_EPS = 1e-10
_KS = 26
_STRIDE = 16
_PAD = 6


def _np_consts(n_h, n_w):
    x = np.arange(_PS, dtype=np.float64) - _PS // 2
    sigma = _PS / math.sqrt(2.0)
    g = np.exp(-(x * x) / (2.0 * sigma * sigma))
    g = g / g.sum()

    ks2 = _KS / 2.0
    xc2 = ks2 - np.abs(np.arange(_KS, dtype=np.float64) + 0.5 - ks2)
    w1 = xc2 / ks2
    wmat = np.zeros((_NB_SP, _PS))
    for s in range(_NB_SP):
        start = s * _STRIDE - _PAD
        for k in range(_KS):
            j = start + k
            if 0 <= j < _PS:
                wmat[s, j] = w1[k]
    wcm = wmat.T * (0.5 * g[:, None])
    wc_big = np.zeros((n_w * _PS, n_w * _NB_SP))
    for p in range(n_w):
        wc_big[p * _PS:(p + 1) * _PS, p * _NB_SP:(p + 1) * _NB_SP] = wcm
    wrm = wmat * g[None, :]
    wr2 = np.zeros((2 * _NB_SP, 2 * _PS))
    for p in range(2):
        wr2[p * _NB_SP:(p + 1) * _NB_SP, p * _PS:(p + 1) * _PS] = wrm
    return wc_big.astype(np.float32), wr2.astype(np.float32)


_CHUNK_PATCH_ROWS = 2
_CHUNK = _CHUNK_PATCH_ROWS * _PS


def _sift_body(x_ref, wc_ref, wr_ref, o_ref):
    n_h, n_w = o_ref.shape[2], o_ref.shape[3]
    wc = wc_ref[...]
    wr2 = wr_ref[...]
    n_ch = (n_h * _PS) // _CHUNK
    parts = []
    for ch in range(n_ch):
        parts.append(_chunk_pool(
            x_ref[0, 0, ch * _CHUNK:(ch + 1) * _CHUNK, :], wc, wr2))
    qa = jnp.concatenate(
        [jnp.stack([p[a] for p in parts]).reshape(n_ch * _CHUNK_PATCH_ROWS
                                                  * _NB_SP, _NB_SP * n_w)
         for a in range(_NB_ANG)], axis=0).reshape(_NB_ANG, n_h * _NB_SP,
                                                   _NB_SP * n_w)
    _finish(qa, n_h, n_w, o_ref)


def _chunk_pool(x, wc, wr2):
    hh, w = x.shape

    col = jax.lax.broadcasted_iota(jnp.int32, (hh, w), 1)
    jj = jax.lax.rem(col, _PS)
    row = jax.lax.broadcasted_iota(jnp.int32, (hh, w), 0)
    ii = jax.lax.rem(row, _PS)
    xr = jnp.concatenate([x[:, 1:], x[:, -1:]], axis=1)
    xl = jnp.concatenate([x[:, :1], x[:, :-1]], axis=1)
    right_val = jnp.where(jj == _PS - 1, x, xr)
    left_val = jnp.where(jj == 0, x, xl)
    gx = right_val - left_val
    xd = jnp.concatenate([x[1:, :], x[-1:, :]], axis=0)
    xu = jnp.concatenate([x[:1, :], x[:-1, :]], axis=0)
    down_val = jnp.where(ii == _PS - 1, x, xd)
    up_val = jnp.where(ii == 0, x, xu)
    gy = down_val - up_val

    mag = jnp.sqrt(gx * gx + gy * gy + 4.0 * _EPS)

    gxe = gx + 2.0 * _EPS
    ax = jnp.abs(gxe)
    ay = jnp.abs(gy)
    mn = jnp.minimum(ax, ay)
    mx = jnp.maximum(ax, ay)
    r = mn / jnp.maximum(mx, 1e-30)
    r2 = r * r
    t = r * (1.2730840300
             + r2 * (-0.4207247425
                     + r2 * (0.2299685627
                             + r2 * (-0.1092053987
                                     + r2 * 0.0268949366))))
    a1 = jnp.where(ay > ax, 2.0 - t, t)
    a2 = jnp.where(gxe < 0.0, 4.0 - a1, a1)
    u8 = jnp.where(gy < 0.0, 8.0 - a2, a2)
    bo0f = jnp.floor(u8)
    wo1 = u8 - bo0f
    b0 = jnp.where(bo0f >= _NB_ANG, bo0f - _NB_ANG, bo0f)
    c1 = wo1 * mag
    c0 = mag - c1

    b0h = b0.astype(jnp.bfloat16)
    c0h = c0.astype(jnp.bfloat16)
    c1h = c1.astype(jnp.bfloat16)
    zero = jnp.zeros_like(c0h)
    eq = [b0h == jnp.bfloat16(float(a)) for a in range(_NB_ANG)]
    out = []
    for a in range(_NB_ANG):
        am1 = (a - 1) % _NB_ANG
        contrib = jnp.where(eq[a], c0h, jnp.where(eq[am1], c1h, zero))
        rp = jnp.dot(wr2, contrib, preferred_element_type=jnp.float32)
        out.append(jnp.dot(rp.astype(jnp.bfloat16), wc,
                           preferred_element_type=jnp.float32))
    return out


def _finish(qa, n_h, n_w, o_ref):
    qa = qa.reshape(_NB_ANG, n_h, _NB_SP, _NB_SP * n_w)
    q = jnp.transpose(qa, (1, 0, 2, 3)).reshape(n_h, _NB_ANG * _NB_SP,
                                                _NB_SP * n_w)
    qt = jnp.transpose(q, (0, 2, 1))
    qt = qt.reshape(n_h, n_w, _NB_SP, 32)
    cat = jnp.concatenate([qt[:, :, k, :] for k in range(_NB_SP)],
                          axis=-1)
    lane = jax.lax.broadcasted_iota(jnp.int32, (n_h, n_w, 128), 2)
    perm = (lane % _NB_SP) * 32 + lane // _NB_SP
    d = jnp.take_along_axis(cat, perm, axis=-1)

    ssq = jnp.sum(d * d, axis=-1, keepdims=True)
    d = d / jnp.maximum(jnp.sqrt(ssq), 1e-12)
    d = jnp.clip(d, 0.0, _CLIP)
    ssq = jnp.sum(d * d, axis=-1, keepdims=True)
    d = d / jnp.maximum(jnp.sqrt(ssq), 1e-12)
    o_ref[0, 0] = d


def kernel(image_batch):
    b, c, h, w = image_batch.shape
    n_h, n_w = h // _PS, w // _PS
    m = _NB_SP * n_w

    wc_big, wr_big = _np_consts(n_h, n_w)
    wc_big = jnp.asarray(wc_big).astype(jnp.bfloat16)
    wr_big = jnp.asarray(wr_big).astype(jnp.bfloat16)

    out = pl.pallas_call(
        _sift_body,
        grid=(b, c),
        in_specs=[
            pl.BlockSpec((1, 1, h, w), lambda i, ci: (i, ci, 0, 0)),
            pl.BlockSpec((w, m), lambda i, ci: (0, 0)),
            pl.BlockSpec((2 * _NB_SP, 2 * _PS), lambda i, ci: (0, 0)),
        ],
        out_specs=pl.BlockSpec((1, 1, n_h, n_w, 128),
                               lambda i, ci: (i, ci, 0, 0, 0)),
        out_shape=jax.ShapeDtypeStruct((b, c, n_h, n_w, 128), jnp.float32),
        compiler_params=pltpu.CompilerParams(
            dimension_semantics=("parallel", "parallel"),
        ),
        name="sift_descriptor",
    )(image_batch, wc_big, wr_big)

    return out.reshape(b * n_h * n_w, c, _NB_ANG * _NB_SP * _NB_SP)


# speedup vs baseline: 1.4933x; 1.4933x over previous
import math

import jax
import jax.numpy as jnp
import numpy as np
from jax.experimental import pallas as pl
from jax.experimental.pallas import tpu as pltpu

_PS = 65
_NB_ANG = 8
_NB_SP = 4
_CLIP = 0.2
_EPS = 1e-10
_KS = 26
_STRIDE = 16
_PAD = 6


def _np_consts(n_h, n_w):
    x = np.arange(_PS, dtype=np.float64) - _PS // 2
    sigma = _PS / math.sqrt(2.0)
    g = np.exp(-(x * x) / (2.0 * sigma * sigma))
    g = g / g.sum()

    ks2 = _KS / 2.0
    xc2 = ks2 - np.abs(np.arange(_KS, dtype=np.float64) + 0.5 - ks2)
    w1 = xc2 / ks2
    wmat = np.zeros((_NB_SP, _PS))
    for s in range(_NB_SP):
        start = s * _STRIDE - _PAD
        for k in range(_KS):
            j = start + k
            if 0 <= j < _PS:
                wmat[s, j] = w1[k]
    wcm = wmat.T * (0.5 * g[:, None])
    wc_big = np.zeros((n_w * _PS, n_w * _NB_SP))
    for p in range(n_w):
        wc_big[p * _PS:(p + 1) * _PS, p * _NB_SP:(p + 1) * _NB_SP] = wcm
    wrm = wmat * g[None, :]
    wr_big = np.zeros((n_h * _NB_SP, n_h * _PS))
    for p in range(n_h):
        wr_big[p * _NB_SP:(p + 1) * _NB_SP, p * _PS:(p + 1) * _PS] = wrm
    return wc_big.astype(np.float32), wr_big.astype(np.float32)


def _sift_body(x_ref, wc_ref, wr_ref, o_ref):
    n_h, n_w = o_ref.shape[2], o_ref.shape[3]
    wc = wc_ref[...]
    wrb = wr_ref[...]
    qa = jnp.stack(_grid_pool(x_ref[0, 0], wc, wrb))
    _finish(qa, n_h, n_w, o_ref)


def _grid_pool(x, wc, wrb):
    hh, w = x.shape

    col = jax.lax.broadcasted_iota(jnp.int32, (hh, w), 1)
    jj = jax.lax.rem(col, _PS)
    row = jax.lax.broadcasted_iota(jnp.int32, (hh, w), 0)
    ii = jax.lax.rem(row, _PS)
    xr = jnp.concatenate([x[:, 1:], x[:, -1:]], axis=1)
    xl = jnp.concatenate([x[:, :1], x[:, :-1]], axis=1)
    right_val = jnp.where(jj == _PS - 1, x, xr)
    left_val = jnp.where(jj == 0, x, xl)
    gx = right_val - left_val
    xd = jnp.concatenate([x[1:, :], x[-1:, :]], axis=0)
    xu = jnp.concatenate([x[:1, :], x[:-1, :]], axis=0)
    down_val = jnp.where(ii == _PS - 1, x, xd)
    up_val = jnp.where(ii == 0, x, xu)
    gy = down_val - up_val

    mag = jnp.sqrt(gx * gx + gy * gy + 4.0 * _EPS)

    gxe = gx + 2.0 * _EPS
    ax = jnp.abs(gxe)
    ay = jnp.abs(gy)
    mn = jnp.minimum(ax, ay)
    mx = jnp.maximum(ax, ay)
    r = mn / jnp.maximum(mx, 1e-30)
    r2 = r * r
    t = r * (1.2730840300
             + r2 * (-0.4207247425
                     + r2 * (0.2299685627
                             + r2 * (-0.1092053987
                                     + r2 * 0.0268949366))))
    a1 = jnp.where(ay > ax, 2.0 - t, t)
    a2 = jnp.where(gxe < 0.0, 4.0 - a1, a1)
    u8 = jnp.where(gy < 0.0, 8.0 - a2, a2)
    bo0f = jnp.floor(u8)
    wo1 = u8 - bo0f
    b0 = jnp.where(bo0f >= _NB_ANG, bo0f - _NB_ANG, bo0f)
    c1 = wo1 * mag
    c0 = mag - c1

    b0h = b0.astype(jnp.bfloat16)
    c0h = c0.astype(jnp.bfloat16)
    c1h = c1.astype(jnp.bfloat16)
    zero = jnp.zeros_like(c0h)
    eq = [b0h == jnp.bfloat16(float(a)) for a in range(_NB_ANG)]
    out = []
    for a in range(_NB_ANG):
        am1 = (a - 1) % _NB_ANG
        contrib = jnp.where(eq[a], c0h, jnp.where(eq[am1], c1h, zero))
        rp = jnp.dot(wrb, contrib, preferred_element_type=jnp.float32)
        out.append(jnp.dot(rp.astype(jnp.bfloat16), wc,
                           preferred_element_type=jnp.float32))
    return out


def _finish(qa, n_h, n_w, o_ref):
    qa = qa.reshape(_NB_ANG, n_h, _NB_SP, _NB_SP * n_w)
    q = jnp.transpose(qa, (1, 0, 2, 3)).reshape(n_h, _NB_ANG * _NB_SP,
                                                _NB_SP * n_w)
    qt = jnp.transpose(q, (0, 2, 1))
    qt = qt.reshape(n_h, n_w, _NB_SP, 32)
    cat = jnp.concatenate([qt[:, :, k, :] for k in range(_NB_SP)],
                          axis=-1)
    lane = jax.lax.broadcasted_iota(jnp.int32, (n_h, n_w, 128), 2)
    perm = (lane % _NB_SP) * 32 + lane // _NB_SP
    d = jnp.take_along_axis(cat, perm, axis=-1)

    ssq = jnp.sum(d * d, axis=-1, keepdims=True)
    d = d / jnp.maximum(jnp.sqrt(ssq), 1e-12)
    d = jnp.clip(d, 0.0, _CLIP)
    ssq = jnp.sum(d * d, axis=-1, keepdims=True)
    d = d / jnp.maximum(jnp.sqrt(ssq), 1e-12)
    o_ref[0, 0] = d


def kernel(image_batch):
    b, c, h, w = image_batch.shape
    n_h, n_w = h // _PS, w // _PS
    m = _NB_SP * n_w

    wc_big, wr_big = _np_consts(n_h, n_w)
    wc_big = jnp.asarray(wc_big).astype(jnp.bfloat16)
    wr_big = jnp.asarray(wr_big).astype(jnp.bfloat16)

    out = pl.pallas_call(
        _sift_body,
        grid=(b, c),
        in_specs=[
            pl.BlockSpec((1, 1, h, w), lambda i, ci: (i, ci, 0, 0)),
            pl.BlockSpec((w, m), lambda i, ci: (0, 0)),
            pl.BlockSpec((_NB_SP * n_h, h), lambda i, ci: (0, 0)),
        ],
        out_specs=pl.BlockSpec((1, 1, n_h, n_w, 128),
                               lambda i, ci: (i, ci, 0, 0, 0)),
        out_shape=jax.ShapeDtypeStruct((b, c, n_h, n_w, 128), jnp.float32),
        compiler_params=pltpu.CompilerParams(
            dimension_semantics=("parallel", "parallel"),
        ),
        name="sift_descriptor",
    )(image_batch, wc_big, wr_big)

    return out.reshape(b * n_h * n_w, c, _NB_ANG * _NB_SP * _NB_SP)


# speedup vs baseline: 1.7682x; 1.1841x over previous
import math

import jax
import jax.numpy as jnp
import numpy as np
from jax.experimental import pallas as pl
from jax.experimental.pallas import tpu as pltpu

_PS = 65
_NB_ANG = 8
_NB_SP = 4
_CLIP = 0.2
_EPS = 1e-10
_KS = 26
_STRIDE = 16
_PAD = 6


def _np_consts(n_h, n_w):
    x = np.arange(_PS, dtype=np.float64) - _PS // 2
    sigma = _PS / math.sqrt(2.0)
    g = np.exp(-(x * x) / (2.0 * sigma * sigma))
    g = g / g.sum()

    ks2 = _KS / 2.0
    xc2 = ks2 - np.abs(np.arange(_KS, dtype=np.float64) + 0.5 - ks2)
    w1 = xc2 / ks2
    wmat = np.zeros((_NB_SP, _PS))
    for s in range(_NB_SP):
        start = s * _STRIDE - _PAD
        for k in range(_KS):
            j = start + k
            if 0 <= j < _PS:
                wmat[s, j] = w1[k]
    wcm = wmat.T * (0.5 * g[:, None])
    wc_big = np.zeros((n_w * _PS, n_w * _NB_SP))
    for p in range(n_w):
        wc_big[p * _PS:(p + 1) * _PS, p * _NB_SP:(p + 1) * _NB_SP] = wcm
    wrm = wmat * g[None, :]
    wr_big = np.zeros((n_h * _NB_SP, n_h * _PS))
    for p in range(n_h):
        wr_big[p * _NB_SP:(p + 1) * _NB_SP, p * _PS:(p + 1) * _PS] = wrm
    return wc_big.astype(np.float32), wr_big.astype(np.float32)


def _sift_body(x_ref, wc_ref, wr_ref, o_ref):
    n_h, n_w = o_ref.shape[2], o_ref.shape[3]
    wc = wc_ref[...]
    wrb = wr_ref[...]
    for ci in range(o_ref.shape[1]):
        qa = jnp.stack(_grid_pool(x_ref[0, ci], wc, wrb))
        _finish(qa, n_h, n_w, o_ref.at[0, ci])


def _grid_pool(x, wc, wrb):
    hh, w = x.shape

    col = jax.lax.broadcasted_iota(jnp.int32, (hh, w), 1)
    jj = jax.lax.rem(col, _PS)
    row = jax.lax.broadcasted_iota(jnp.int32, (hh, w), 0)
    ii = jax.lax.rem(row, _PS)
    xr = jnp.concatenate([x[:, 1:], x[:, -1:]], axis=1)
    xl = jnp.concatenate([x[:, :1], x[:, :-1]], axis=1)
    right_val = jnp.where(jj == _PS - 1, x, xr)
    left_val = jnp.where(jj == 0, x, xl)
    gx = right_val - left_val
    xd = jnp.concatenate([x[1:, :], x[-1:, :]], axis=0)
    xu = jnp.concatenate([x[:1, :], x[:-1, :]], axis=0)
    down_val = jnp.where(ii == _PS - 1, x, xd)
    up_val = jnp.where(ii == 0, x, xu)
    gy = down_val - up_val

    mag = jnp.sqrt(gx * gx + gy * gy + 4.0 * _EPS)

    gxe = gx + 2.0 * _EPS
    ax = jnp.abs(gxe)
    ay = jnp.abs(gy)
    mn = jnp.minimum(ax, ay)
    mx = jnp.maximum(ax, ay)
    r = mn / jnp.maximum(mx, 1e-30)
    r2 = r * r
    t = r * (1.2730840300
             + r2 * (-0.4207247425
                     + r2 * (0.2299685627
                             + r2 * (-0.1092053987
                                     + r2 * 0.0268949366))))
    a1 = jnp.where(ay > ax, 2.0 - t, t)
    a2 = jnp.where(gxe < 0.0, 4.0 - a1, a1)
    u8 = jnp.where(gy < 0.0, 8.0 - a2, a2)
    bo0f = jnp.floor(u8)
    wo1 = u8 - bo0f
    b0 = jnp.where(bo0f >= _NB_ANG, bo0f - _NB_ANG, bo0f)
    c1 = wo1 * mag
    c0 = mag - c1

    b0h = b0.astype(jnp.bfloat16)
    c0h = c0.astype(jnp.bfloat16)
    c1h = c1.astype(jnp.bfloat16)
    zero = jnp.zeros_like(c0h)
    eq = [b0h == jnp.bfloat16(float(a)) for a in range(_NB_ANG)]
    out = []
    for a in range(_NB_ANG):
        am1 = (a - 1) % _NB_ANG
        contrib = jnp.where(eq[a], c0h, jnp.where(eq[am1], c1h, zero))
        rp = jnp.dot(wrb, contrib, preferred_element_type=jnp.float32)
        out.append(jnp.dot(rp.astype(jnp.bfloat16), wc,
                           preferred_element_type=jnp.float32))
    return out


def _finish(qa, n_h, n_w, o_ref):
    qa = qa.reshape(_NB_ANG, n_h, _NB_SP, _NB_SP * n_w)
    q = jnp.transpose(qa, (1, 0, 2, 3)).reshape(n_h, _NB_ANG * _NB_SP,
                                                _NB_SP * n_w)
    qt = jnp.transpose(q, (0, 2, 1))
    qt = qt.reshape(n_h, n_w, _NB_SP, 32)
    cat = jnp.concatenate([qt[:, :, k, :] for k in range(_NB_SP)],
                          axis=-1)
    lane = jax.lax.broadcasted_iota(jnp.int32, (n_h, n_w, 128), 2)
    perm = (lane % _NB_SP) * 32 + lane // _NB_SP
    d = jnp.take_along_axis(cat, perm, axis=-1)

    ssq = jnp.sum(d * d, axis=-1, keepdims=True)
    d = d / jnp.maximum(jnp.sqrt(ssq), 1e-12)
    d = jnp.clip(d, 0.0, _CLIP)
    ssq = jnp.sum(d * d, axis=-1, keepdims=True)
    d = d / jnp.maximum(jnp.sqrt(ssq), 1e-12)
    o_ref[...] = d


def kernel(image_batch):
    b, c, h, w = image_batch.shape
    n_h, n_w = h // _PS, w // _PS
    m = _NB_SP * n_w

    wc_big, wr_big = _np_consts(n_h, n_w)
    wc_big = jnp.asarray(wc_big).astype(jnp.bfloat16)
    wr_big = jnp.asarray(wr_big).astype(jnp.bfloat16)

    out = pl.pallas_call(
        _sift_body,
        grid=(b,),
        in_specs=[
            pl.BlockSpec((1, c, h, w), lambda i: (i, 0, 0, 0)),
            pl.BlockSpec((w, m), lambda i: (0, 0)),
            pl.BlockSpec((_NB_SP * n_h, h), lambda i: (0, 0)),
        ],
        out_specs=pl.BlockSpec((1, c, n_h, n_w, 128),
                               lambda i: (i, 0, 0, 0, 0)),
        out_shape=jax.ShapeDtypeStruct((b, c, n_h, n_w, 128), jnp.float32),
        compiler_params=pltpu.CompilerParams(
            dimension_semantics=("parallel",),
        ),
        name="sift_descriptor",
    )(image_batch, wc_big, wr_big)

    return out.reshape(b * n_h * n_w, c, _NB_ANG * _NB_SP * _NB_SP)


# speedup vs baseline: 1.7998x; 1.0179x over previous
import math

import jax
import jax.numpy as jnp
import numpy as np
from jax.experimental import pallas as pl
from jax.experimental.pallas import tpu as pltpu

_PS = 65
_NB_ANG = 8
_NB_SP = 4
_CLIP = 0.2
_EPS = 1e-10
_KS = 26
_STRIDE = 16
_PAD = 6


def _np_consts(n_h, n_w):
    x = np.arange(_PS, dtype=np.float64) - _PS // 2
    sigma = _PS / math.sqrt(2.0)
    g = np.exp(-(x * x) / (2.0 * sigma * sigma))
    g = g / g.sum()

    ks2 = _KS / 2.0
    xc2 = ks2 - np.abs(np.arange(_KS, dtype=np.float64) + 0.5 - ks2)
    w1 = xc2 / ks2
    wmat = np.zeros((_NB_SP, _PS))
    for s in range(_NB_SP):
        start = s * _STRIDE - _PAD
        for k in range(_KS):
            j = start + k
            if 0 <= j < _PS:
                wmat[s, j] = w1[k]
    wcm = wmat.T * (0.5 * g[:, None])
    wc_big = np.zeros((n_w * _PS, n_w * _NB_SP))
    for p in range(n_w):
        wc_big[p * _PS:(p + 1) * _PS, p * _NB_SP:(p + 1) * _NB_SP] = wcm
    wrm = wmat * g[None, :]
    wr_big = np.zeros((n_h * _NB_SP, n_h * _PS))
    for p in range(n_h):
        wr_big[p * _NB_SP:(p + 1) * _NB_SP, p * _PS:(p + 1) * _PS] = wrm
    return wc_big.astype(np.float32), wr_big.astype(np.float32)


def _sift_body(x_ref, wc_ref, wr_ref, o_ref):
    n_h, n_w = o_ref.shape[2], o_ref.shape[3]
    wc = wc_ref[...]
    wrb = wr_ref[...]
    for ci in range(o_ref.shape[1]):
        qa = jnp.stack(_grid_pool(x_ref[0, ci], wc, wrb))
        _finish(qa, n_h, n_w, o_ref.at[0, ci])


def _grid_pool(x, wc, wrb):
    hh, w = x.shape

    col = jax.lax.broadcasted_iota(jnp.int32, (hh, w), 1)
    jj = jax.lax.rem(col, _PS)
    row = jax.lax.broadcasted_iota(jnp.int32, (hh, w), 0)
    ii = jax.lax.rem(row, _PS)
    xr = jnp.concatenate([x[:, 1:], x[:, -1:]], axis=1)
    xl = jnp.concatenate([x[:, :1], x[:, :-1]], axis=1)
    right_val = jnp.where(jj == _PS - 1, x, xr)
    left_val = jnp.where(jj == 0, x, xl)
    gx = right_val - left_val
    xd = jnp.concatenate([x[1:, :], x[-1:, :]], axis=0)
    xu = jnp.concatenate([x[:1, :], x[:-1, :]], axis=0)
    down_val = jnp.where(ii == _PS - 1, x, xd)
    up_val = jnp.where(ii == 0, x, xu)
    gy = down_val - up_val

    mag = jnp.sqrt(gx * gx + gy * gy + 4.0 * _EPS)

    gxe = gx + 2.0 * _EPS
    ax = jnp.abs(gxe)
    ay = jnp.abs(gy)
    mn = jnp.minimum(ax, ay)
    mx = jnp.maximum(ax, ay)
    r = mn / jnp.maximum(mx, 1e-30)
    r2 = r * r
    t = r * (1.2730840300
             + r2 * (-0.4207247425
                     + r2 * (0.2299685627
                             + r2 * (-0.1092053987
                                     + r2 * 0.0268949366))))
    swap = ay > ax
    xneg = gxe < 0.0
    yneg = gy < 0.0
    flip = swap ^ xneg ^ yneg
    wo1 = jnp.where(flip, 1.0 - t, t)
    kpos = jnp.where(xneg, jnp.where(swap, 2.0, 3.0),
                     jnp.where(swap, 1.0, 0.0))
    b0 = jnp.where(yneg, 7.0 - kpos, kpos)
    c1 = wo1 * mag
    c0 = mag - c1

    b0h = b0.astype(jnp.bfloat16)
    c0h = c0.astype(jnp.bfloat16)
    c1h = c1.astype(jnp.bfloat16)
    zero = jnp.zeros_like(c0h)
    eq = [b0h == jnp.bfloat16(float(a)) for a in range(_NB_ANG)]
    out = []
    for a in range(_NB_ANG):
        am1 = (a - 1) % _NB_ANG
        contrib = jnp.where(eq[a], c0h, jnp.where(eq[am1], c1h, zero))
        rp = jnp.dot(wrb, contrib, preferred_element_type=jnp.float32)
        out.append(jnp.dot(rp.astype(jnp.bfloat16), wc,
                           preferred_element_type=jnp.float32))
    return out


def _finish(qa, n_h, n_w, o_ref):
    qa = qa.reshape(_NB_ANG, n_h, _NB_SP, _NB_SP * n_w)
    q = jnp.transpose(qa, (1, 0, 2, 3)).reshape(n_h, _NB_ANG * _NB_SP,
                                                _NB_SP * n_w)
    qt = jnp.transpose(q, (0, 2, 1))
    qt = qt.reshape(n_h, n_w, _NB_SP, 32)
    cat = jnp.concatenate([qt[:, :, k, :] for k in range(_NB_SP)],
                          axis=-1)
    lane = jax.lax.broadcasted_iota(jnp.int32, (n_h, n_w, 128), 2)
    perm = (lane % _NB_SP) * 32 + lane // _NB_SP
    d = jnp.take_along_axis(cat, perm, axis=-1)

    ssq = jnp.sum(d * d, axis=-1, keepdims=True)
    d = d / jnp.maximum(jnp.sqrt(ssq), 1e-12)
    d = jnp.clip(d, 0.0, _CLIP)
    ssq = jnp.sum(d * d, axis=-1, keepdims=True)
    d = d / jnp.maximum(jnp.sqrt(ssq), 1e-12)
    o_ref[...] = d


def kernel(image_batch):
    b, c, h, w = image_batch.shape
    n_h, n_w = h // _PS, w // _PS
    m = _NB_SP * n_w

    wc_big, wr_big = _np_consts(n_h, n_w)
    wc_big = jnp.asarray(wc_big).astype(jnp.bfloat16)
    wr_big = jnp.asarray(wr_big).astype(jnp.bfloat16)

    out = pl.pallas_call(
        _sift_body,
        grid=(b,),
        in_specs=[
            pl.BlockSpec((1, c, h, w), lambda i: (i, 0, 0, 0)),
            pl.BlockSpec((w, m), lambda i: (0, 0)),
            pl.BlockSpec((_NB_SP * n_h, h), lambda i: (0, 0)),
        ],
        out_specs=pl.BlockSpec((1, c, n_h, n_w, 128),
                               lambda i: (i, 0, 0, 0, 0)),
        out_shape=jax.ShapeDtypeStruct((b, c, n_h, n_w, 128), jnp.float32),
        compiler_params=pltpu.CompilerParams(
            dimension_semantics=("parallel",),
        ),
        name="sift_descriptor",
    )(image_batch, wc_big, wr_big)

    return out.reshape(b * n_h * n_w, c, _NB_ANG * _NB_SP * _NB_SP)
